# baseline (speedup 1.0000x reference)
import functools

import jax
import jax.numpy as jnp
from jax import lax
from jax.experimental import pallas as pl
from jax.experimental.pallas import tpu as pltpu
from jax.experimental.pallas import tpu_sc as plsc

N_NODES = 10000
N_PAD = 10112
E_EDGES = 320000
CHUNK = 128
N_CHUNKS = 2560
EP = N_CHUNKS * CHUNK
D = 128
N_SC = 2
N_SUB = 16
STAGES = 2
C0 = 2048
CPT0 = C0 // N_SUB
CPT1 = (N_CHUNKS - C0) // N_SUB
SS0 = CPT0 // STAGES
SS1 = CPT1 // STAGES
N_CHUNKS_PAD = N_CHUNKS + SS0
ROWS_PER_TILE = N_PAD // N_SUB
NC_PAD = 10240
CNT_PER_TILE = NC_PAD // N_SUB


def _sc_agg_body(do_cnt, *refs):
    if do_cnt:
        (h_hbm, srcm, dstm, agg_out, cnt_out,
         src_v, dst_v, rows_v, ones_v, agg_sh, cnt_sh, sem) = refs
    else:
        (h_hbm, srcm, dstm, agg_out,
         src_v, dst_v, rows_v, agg_sh, sem) = refs
        cnt_out = cnt_sh = ones_v = None
    c = lax.axis_index("c")
    s = lax.axis_index("s")

    def zrow(i, carry):
        for k in range(D // 16):
            rows_v[i, pl.ds(k * 16, 16)] = jnp.zeros((16,), jnp.float32)
        return carry
    lax.fori_loop(0, CHUNK, zrow, 0)
    if do_cnt:
        for k in range(CHUNK // 16):
            ones_v[pl.ds(k * 16, 16)] = jnp.ones((16,), jnp.float32)

    zbase = s * ROWS_PER_TILE
    TAIL = ROWS_PER_TILE - (ROWS_PER_TILE // CHUNK) * CHUNK
    for k in range(ROWS_PER_TILE // CHUNK):
        pltpu.sync_copy(rows_v.at[pl.ds(0, CHUNK)],
                        agg_sh.at[pl.ds(zbase + k * CHUNK, CHUNK)])
    pltpu.sync_copy(rows_v.at[pl.ds(0, TAIL)],
                    agg_sh.at[pl.ds(zbase + ROWS_PER_TILE - TAIL, TAIL)])
    if do_cnt:
        for k in range(CNT_PER_TILE // CHUNK):
            pltpu.sync_copy(rows_v.at[0],
                            cnt_sh.at[pl.ds(s * CNT_PER_TILE + k * CHUNK, CHUNK)])
    plsc.subcore_barrier()

    base_chunk = jnp.where(c == 0, s * CPT0, C0 + s * CPT1)
    nst = jnp.where(c == 0, SS0, SS1)
    for stage in range(STAGES):
        sbase = base_chunk + stage * nst
        pltpu.sync_copy(srcm.at[pl.ds(sbase, SS0)], src_v)
        pltpu.sync_copy(dstm.at[pl.ds(sbase, SS0)], dst_v)

        def issue(j, half):
            pltpu.async_copy(h_hbm.at[src_v.at[j, pl.ds(0, 64)]],
                             rows_v.at[pl.ds(half, 64)], sem)
            pltpu.async_copy(h_hbm.at[src_v.at[j, pl.ds(64, 64)]],
                             rows_v.at[pl.ds(half + 64, 64)], sem)

        issue(0, 0)
        issue(1, CHUNK)

        def body(j, carry):
            half = (j % 2) * CHUNK
            for q in range(2):
                pltpu.make_async_copy(h_hbm.at[src_v.at[j, pl.ds(0, 64)]],
                                      rows_v.at[pl.ds(half + q * 64, 64)],
                                      sem).wait()
            pltpu.sync_copy(rows_v.at[pl.ds(half, CHUNK)],
                            agg_sh.at[dst_v.at[j]], add=True)
            if do_cnt:
                pltpu.sync_copy(ones_v, cnt_sh.at[dst_v.at[j]], add=True)

            @pl.when(j + 2 < nst)
            def _():
                issue(j + 2, half)
            return carry
        lax.fori_loop(0, nst, body, 0)
    plsc.subcore_barrier()

    pltpu.sync_copy(agg_sh.at[pl.ds(zbase, ROWS_PER_TILE)],
                    agg_out.at[c].at[pl.ds(zbase, ROWS_PER_TILE)])
    if do_cnt:
        pltpu.sync_copy(cnt_sh.at[pl.ds(s * CNT_PER_TILE, CNT_PER_TILE)],
                        cnt_out.at[c].at[pl.ds(s * CNT_PER_TILE, CNT_PER_TILE)])


def _make_sc_agg(do_cnt):
    mesh = plsc.VectorSubcoreMesh(core_axis_name="c", subcore_axis_name="s")
    out_type = [jax.ShapeDtypeStruct((N_SC, N_PAD, D), jnp.float32)]
    if do_cnt:
        out_type.append(jax.ShapeDtypeStruct((N_SC, NC_PAD), jnp.float32))
    scratch = [
        pltpu.VMEM((SS0, CHUNK), jnp.int32),
        pltpu.VMEM((SS0, CHUNK), jnp.int32),
        pltpu.VMEM((2 * CHUNK, D), jnp.float32),
    ]
    if do_cnt:
        scratch.append(pltpu.VMEM((CHUNK,), jnp.float32))
    scratch.append(pltpu.VMEM_SHARED((N_PAD, D), jnp.float32))
    if do_cnt:
        scratch.append(pltpu.VMEM_SHARED((NC_PAD,), jnp.float32))
    scratch.append(pltpu.SemaphoreType.DMA)
    return pl.kernel(
        functools.partial(_sc_agg_body, do_cnt),
        out_type=tuple(out_type) if len(out_type) > 1 else out_type[0],
        mesh=mesh,
        scratch_types=tuple(scratch),
    )


_sc_agg_cnt = _make_sc_agg(True)
_sc_agg = _make_sc_agg(False)

BR = 1000


def _dense_body(agg_ref, cnt_ref, h_ref, wl_ref, wr_ref, b_ref, a_ref, out_ref):
    r = agg_ref[0] + agg_ref[1]
    cnt = cnt_ref[0] + cnt_ref[1]
    r = r / jnp.maximum(cnt, 1.0)
    o = (jnp.dot(r, wl_ref[...], preferred_element_type=jnp.float32)
         + b_ref[...]
         + jnp.dot(h_ref[...], wr_ref[...], preferred_element_type=jnp.float32))
    av = a_ref[0, 0]
    out_ref[...] = jnp.where(o >= 0, o, av * o)


_dense = pl.pallas_call(
    _dense_body,
    grid=(N_NODES // BR,),
    in_specs=[
        pl.BlockSpec((N_SC, BR, D), lambda i: (0, i, 0)),
        pl.BlockSpec((N_SC, BR, 1), lambda i: (0, i, 0)),
        pl.BlockSpec((BR, D), lambda i: (i, 0)),
        pl.BlockSpec((D, D), lambda i: (0, 0)),
        pl.BlockSpec((D, D), lambda i: (0, 0)),
        pl.BlockSpec((1, D), lambda i: (0, 0)),
        pl.BlockSpec((1, 1), lambda i: (0, 0)),
    ],
    out_specs=pl.BlockSpec((BR, D), lambda i: (i, 0)),
    out_shape=jax.ShapeDtypeStruct((N_NODES, D), jnp.float32),
)


_dense_win = pl.pallas_call(
    _dense_body,
    grid=(1,),
    in_specs=[
        pl.BlockSpec((N_SC, 1024, D), lambda i: (0, 0, 0)),
        pl.BlockSpec((N_SC, 1024, 1), lambda i: (0, 0, 0)),
        pl.BlockSpec((1024, D), lambda i: (0, 0)),
        pl.BlockSpec((D, D), lambda i: (0, 0)),
        pl.BlockSpec((D, D), lambda i: (0, 0)),
        pl.BlockSpec((1, D), lambda i: (0, 0)),
        pl.BlockSpec((1, 1), lambda i: (0, 0)),
    ],
    out_specs=pl.BlockSpec((1024, D), lambda i: (0, 0)),
    out_shape=jax.ShapeDtypeStruct((1024, D), jnp.float32),
)


def kernel(x, edge_index, W1_l, b1, W1_r, W2_l, b2, W2_r, W3_l, b3, W3_r, a,
           batch_size, layer):
    src = edge_index[0]
    dst = edge_index[1]
    pad = N_CHUNKS_PAD * CHUNK - E_EDGES
    srcm = jnp.concatenate([src, jnp.zeros((pad,), jnp.int32)]).reshape(N_CHUNKS_PAD, CHUNK)
    dstm = jnp.concatenate([dst, jnp.full((pad,), N_NODES, jnp.int32)]).reshape(N_CHUNKS_PAD, CHUNK)
    a2 = jnp.reshape(a, (1, 1)).astype(jnp.float32)

    agg1, cnt = _sc_agg_cnt(x, srcm, dstm)
    cnt3 = cnt[:, :, None]
    h1 = _dense(agg1, cnt3, x, W1_l, W1_r, b1.reshape(1, D), a2)
    agg2 = _sc_agg(h1, srcm, dstm)
    h2 = _dense(agg2, cnt3, h1, W2_l, W2_r, b2.reshape(1, D), a2)
    agg3 = _sc_agg(h2, srcm, dstm)
    lo = jnp.clip(jnp.asarray(batch_size, jnp.int32) - 1024, 0, N_NODES - 1024)
    agg3w = lax.dynamic_slice(agg3, (0, lo, 0), (N_SC, 1024, D))
    cnt3w = lax.dynamic_slice(cnt3, (0, lo, 0), (N_SC, 1024, 1))
    h2w = lax.dynamic_slice(h2, (lo, 0), (1024, D))
    return _dense_win(agg3w, cnt3w, h2w, W3_l, W3_r, b3.reshape(1, D), a2)


# speedup vs baseline: 1.0177x; 1.0177x over previous
import functools

import jax
import jax.numpy as jnp
from jax import lax
from jax.experimental import pallas as pl
from jax.experimental.pallas import tpu as pltpu
from jax.experimental.pallas import tpu_sc as plsc

N_NODES = 10000
N_PAD = 10112
E_EDGES = 320000
CHUNK = 128
N_CHUNKS = 2560
EP = N_CHUNKS * CHUNK
D = 128
N_SC = 2
N_SUB = 16
STAGES = 2
C0 = 2048
CPT0 = C0 // N_SUB
CPT1 = (N_CHUNKS - C0) // N_SUB
SS0 = CPT0 // STAGES
SS1 = CPT1 // STAGES
N_CHUNKS_PAD = N_CHUNKS + SS0
ROWS_PER_TILE = N_PAD // N_SUB
NC_PAD = 10240
CNT_PER_TILE = NC_PAD // N_SUB


def _sc_agg_body(do_cnt, *refs):
    if do_cnt:
        (h_hbm, srcm, dstm, agg_out, cnt_out,
         src_v, dst_v, rows_v, rows2_v, ones_v, agg_sh, cnt_sh, sem) = refs
    else:
        (h_hbm, srcm, dstm, agg_out,
         src_v, dst_v, rows_v, rows2_v, agg_sh, sem) = refs
        cnt_out = cnt_sh = ones_v = None
    c = lax.axis_index("c")
    s = lax.axis_index("s")

    def zrow(i, carry):
        for k in range(D // 16):
            rows_v[i, pl.ds(k * 16, 16)] = jnp.zeros((16,), jnp.float32)
        return carry
    lax.fori_loop(0, CHUNK, zrow, 0)
    if do_cnt:
        for k in range(CHUNK // 16):
            ones_v[pl.ds(k * 16, 16)] = jnp.ones((16,), jnp.float32)

    zbase = s * ROWS_PER_TILE
    TAIL = ROWS_PER_TILE - (ROWS_PER_TILE // CHUNK) * CHUNK
    for k in range(ROWS_PER_TILE // CHUNK):
        pltpu.sync_copy(rows_v, agg_sh.at[pl.ds(zbase + k * CHUNK, CHUNK)])
    pltpu.sync_copy(rows_v.at[pl.ds(0, TAIL)],
                    agg_sh.at[pl.ds(zbase + ROWS_PER_TILE - TAIL, TAIL)])
    if do_cnt:
        for k in range(CNT_PER_TILE // CHUNK):
            pltpu.sync_copy(rows_v.at[0],
                            cnt_sh.at[pl.ds(s * CNT_PER_TILE + k * CHUNK, CHUNK)])
    plsc.subcore_barrier()

    base_chunk = jnp.where(c == 0, s * CPT0, C0 + s * CPT1)
    nst = jnp.where(c == 0, SS0, SS1)
    for stage in range(STAGES):
        sbase = base_chunk + stage * nst
        pltpu.sync_copy(srcm.at[pl.ds(sbase, SS0)], src_v)
        pltpu.sync_copy(dstm.at[pl.ds(sbase, SS0)], dst_v)

        pltpu.async_copy(h_hbm.at[src_v.at[0]], rows_v, sem)
        pltpu.async_copy(h_hbm.at[src_v.at[1]], rows2_v, sem)

        def scatter_pair(j0):
            pltpu.make_async_copy(h_hbm.at[src_v.at[j0]], rows_v, sem).wait()
            pltpu.make_async_copy(h_hbm.at[src_v.at[j0 + 1]], rows2_v, sem).wait()
            pltpu.sync_copy(rows_v, agg_sh.at[dst_v.at[j0]], add=True)
            pltpu.sync_copy(rows2_v, agg_sh.at[dst_v.at[j0 + 1]], add=True)
            if do_cnt:
                pltpu.sync_copy(ones_v, cnt_sh.at[dst_v.at[j0]], add=True)
                pltpu.sync_copy(ones_v, cnt_sh.at[dst_v.at[j0 + 1]], add=True)

        def body(j, carry):
            j0 = 2 * j
            scatter_pair(j0)
            pltpu.async_copy(h_hbm.at[src_v.at[j0 + 2]], rows_v, sem)
            pltpu.async_copy(h_hbm.at[src_v.at[j0 + 3]], rows2_v, sem)
            return carry
        lax.fori_loop(0, nst // 2 - 1, body, 0)
        scatter_pair(nst - 2)
    plsc.subcore_barrier()

    pltpu.sync_copy(agg_sh.at[pl.ds(zbase, ROWS_PER_TILE)],
                    agg_out.at[c].at[pl.ds(zbase, ROWS_PER_TILE)])
    if do_cnt:
        pltpu.sync_copy(cnt_sh.at[pl.ds(s * CNT_PER_TILE, CNT_PER_TILE)],
                        cnt_out.at[c].at[pl.ds(s * CNT_PER_TILE, CNT_PER_TILE)])


def _make_sc_agg(do_cnt):
    mesh = plsc.VectorSubcoreMesh(core_axis_name="c", subcore_axis_name="s")
    out_type = [jax.ShapeDtypeStruct((N_SC, N_PAD, D), jnp.float32)]
    if do_cnt:
        out_type.append(jax.ShapeDtypeStruct((N_SC, NC_PAD), jnp.float32))
    scratch = [
        pltpu.VMEM((SS0, CHUNK), jnp.int32),
        pltpu.VMEM((SS0, CHUNK), jnp.int32),
        pltpu.VMEM((CHUNK, D), jnp.float32),
        pltpu.VMEM((CHUNK, D), jnp.float32),
    ]
    if do_cnt:
        scratch.append(pltpu.VMEM((CHUNK,), jnp.float32))
    scratch.append(pltpu.VMEM_SHARED((N_PAD, D), jnp.float32))
    if do_cnt:
        scratch.append(pltpu.VMEM_SHARED((NC_PAD,), jnp.float32))
    scratch.append(pltpu.SemaphoreType.DMA)
    return pl.kernel(
        functools.partial(_sc_agg_body, do_cnt),
        out_type=tuple(out_type) if len(out_type) > 1 else out_type[0],
        mesh=mesh,
        scratch_types=tuple(scratch),
    )


_sc_agg_cnt = _make_sc_agg(True)
_sc_agg = _make_sc_agg(False)

BR = 1000


def _dense_body(agg_ref, cnt_ref, h_ref, wl_ref, wr_ref, b_ref, a_ref, out_ref):
    r = agg_ref[0] + agg_ref[1]
    cnt = cnt_ref[0] + cnt_ref[1]
    r = r / jnp.maximum(cnt, 1.0)
    o = (jnp.dot(r, wl_ref[...], preferred_element_type=jnp.float32)
         + b_ref[...]
         + jnp.dot(h_ref[...], wr_ref[...], preferred_element_type=jnp.float32))
    av = a_ref[0, 0]
    out_ref[...] = jnp.where(o >= 0, o, av * o)


_dense = pl.pallas_call(
    _dense_body,
    grid=(N_NODES // BR,),
    in_specs=[
        pl.BlockSpec((N_SC, BR, D), lambda i: (0, i, 0)),
        pl.BlockSpec((N_SC, BR, 1), lambda i: (0, i, 0)),
        pl.BlockSpec((BR, D), lambda i: (i, 0)),
        pl.BlockSpec((D, D), lambda i: (0, 0)),
        pl.BlockSpec((D, D), lambda i: (0, 0)),
        pl.BlockSpec((1, D), lambda i: (0, 0)),
        pl.BlockSpec((1, 1), lambda i: (0, 0)),
    ],
    out_specs=pl.BlockSpec((BR, D), lambda i: (i, 0)),
    out_shape=jax.ShapeDtypeStruct((N_NODES, D), jnp.float32),
)


_dense_win = pl.pallas_call(
    _dense_body,
    grid=(1,),
    in_specs=[
        pl.BlockSpec((N_SC, 1024, D), lambda i: (0, 0, 0)),
        pl.BlockSpec((N_SC, 1024, 1), lambda i: (0, 0, 0)),
        pl.BlockSpec((1024, D), lambda i: (0, 0)),
        pl.BlockSpec((D, D), lambda i: (0, 0)),
        pl.BlockSpec((D, D), lambda i: (0, 0)),
        pl.BlockSpec((1, D), lambda i: (0, 0)),
        pl.BlockSpec((1, 1), lambda i: (0, 0)),
    ],
    out_specs=pl.BlockSpec((1024, D), lambda i: (0, 0)),
    out_shape=jax.ShapeDtypeStruct((1024, D), jnp.float32),
)


def kernel(x, edge_index, W1_l, b1, W1_r, W2_l, b2, W2_r, W3_l, b3, W3_r, a,
           batch_size, layer):
    src = edge_index[0]
    dst = edge_index[1]
    pad = N_CHUNKS_PAD * CHUNK - E_EDGES
    srcm = jnp.concatenate([src, jnp.zeros((pad,), jnp.int32)]).reshape(N_CHUNKS_PAD, CHUNK)
    dstm = jnp.concatenate([dst, jnp.full((pad,), N_NODES, jnp.int32)]).reshape(N_CHUNKS_PAD, CHUNK)
    a2 = jnp.reshape(a, (1, 1)).astype(jnp.float32)

    agg1, cnt = _sc_agg_cnt(x, srcm, dstm)
    cnt3 = cnt[:, :, None]
    h1 = _dense(agg1, cnt3, x, W1_l, W1_r, b1.reshape(1, D), a2)
    agg2 = _sc_agg(h1, srcm, dstm)
    h2 = _dense(agg2, cnt3, h1, W2_l, W2_r, b2.reshape(1, D), a2)
    agg3 = _sc_agg(h2, srcm, dstm)
    lo = jnp.clip(jnp.asarray(batch_size, jnp.int32) - 1024, 0, N_NODES - 1024)
    agg3w = lax.dynamic_slice(agg3, (0, lo, 0), (N_SC, 1024, D))
    cnt3w = lax.dynamic_slice(cnt3, (0, lo, 0), (N_SC, 1024, 1))
    h2w = lax.dynamic_slice(h2, (lo, 0), (1024, D))
    return _dense_win(agg3w, cnt3w, h2w, W3_l, W3_r, b3.reshape(1, D), a2)


# speedup vs baseline: 1.0197x; 1.0020x over previous
import functools

import jax
import jax.numpy as jnp
from jax import lax
from jax.experimental import pallas as pl
from jax.experimental.pallas import tpu as pltpu
from jax.experimental.pallas import tpu_sc as plsc

N_NODES = 10000
N_PAD = 10112
E_EDGES = 320000
CHUNK = 128
N_CHUNKS = 2560
EP = N_CHUNKS * CHUNK
D = 128
N_SC = 2
N_SUB = 16
STAGES = 2
C0 = 2048
CPT0 = C0 // N_SUB
CPT1 = (N_CHUNKS - C0) // N_SUB
SS0 = CPT0 // STAGES
SS1 = CPT1 // STAGES
N_CHUNKS_PAD = N_CHUNKS + SS0
ROWS_PER_TILE = N_PAD // N_SUB
NC_PAD = 10240
CNT_PER_TILE = NC_PAD // N_SUB


def _sc_agg_body(do_cnt, *refs):
    if do_cnt:
        (h_hbm, srcm, dstm, agg_out, cnt_out,
         src_v, dst_v, rows_v, rows2_v, ones_v, agg_sh, cnt_sh, sem) = refs
    else:
        (h_hbm, srcm, dstm, agg_out,
         src_v, dst_v, rows_v, rows2_v, agg_sh, sem) = refs
        cnt_out = cnt_sh = ones_v = None
    c = lax.axis_index("c")
    s = lax.axis_index("s")

    def zrow(i, carry):
        for k in range(D // 16):
            rows_v[i, pl.ds(k * 16, 16)] = jnp.zeros((16,), jnp.float32)
        return carry
    lax.fori_loop(0, CHUNK, zrow, 0)
    if do_cnt:
        for k in range(CHUNK // 16):
            ones_v[pl.ds(k * 16, 16)] = jnp.ones((16,), jnp.float32)

    zbase = s * ROWS_PER_TILE
    TAIL = ROWS_PER_TILE - (ROWS_PER_TILE // CHUNK) * CHUNK

    def zcopies(copy):
        for k in range(ROWS_PER_TILE // CHUNK):
            copy(rows_v, agg_sh.at[pl.ds(zbase + k * CHUNK, CHUNK)])
        copy(rows_v.at[pl.ds(0, TAIL)],
             agg_sh.at[pl.ds(zbase + ROWS_PER_TILE - TAIL, TAIL)])
        if do_cnt:
            for k in range(CNT_PER_TILE // CHUNK):
                copy(rows_v.at[0],
                     cnt_sh.at[pl.ds(s * CNT_PER_TILE + k * CHUNK, CHUNK)])

    zcopies(lambda a, b: pltpu.async_copy(a, b, sem))
    zcopies(lambda a, b: pltpu.make_async_copy(a, b, sem).wait())
    plsc.subcore_barrier()

    base_chunk = jnp.where(c == 0, s * CPT0, C0 + s * CPT1)
    nst = jnp.where(c == 0, SS0, SS1)
    for stage in range(STAGES):
        sbase = base_chunk + stage * nst
        pltpu.async_copy(srcm.at[pl.ds(sbase, SS0)], src_v, sem)
        pltpu.async_copy(dstm.at[pl.ds(sbase, SS0)], dst_v, sem)
        pltpu.make_async_copy(srcm.at[pl.ds(sbase, SS0)], src_v, sem).wait()
        pltpu.make_async_copy(dstm.at[pl.ds(sbase, SS0)], dst_v, sem).wait()

        pltpu.async_copy(h_hbm.at[src_v.at[0]], rows_v, sem)
        pltpu.async_copy(h_hbm.at[src_v.at[1]], rows2_v, sem)

        def scatter_pair(j0):
            pltpu.make_async_copy(h_hbm.at[src_v.at[j0]], rows_v, sem).wait()
            pltpu.make_async_copy(h_hbm.at[src_v.at[j0 + 1]], rows2_v, sem).wait()
            pltpu.sync_copy(rows_v, agg_sh.at[dst_v.at[j0]], add=True)
            pltpu.sync_copy(rows2_v, agg_sh.at[dst_v.at[j0 + 1]], add=True)
            if do_cnt:
                pltpu.sync_copy(ones_v, cnt_sh.at[dst_v.at[j0]], add=True)
                pltpu.sync_copy(ones_v, cnt_sh.at[dst_v.at[j0 + 1]], add=True)

        def body(j, carry):
            j0 = 2 * j
            scatter_pair(j0)
            pltpu.async_copy(h_hbm.at[src_v.at[j0 + 2]], rows_v, sem)
            pltpu.async_copy(h_hbm.at[src_v.at[j0 + 3]], rows2_v, sem)
            return carry
        lax.fori_loop(0, nst // 2 - 1, body, 0)
        scatter_pair(nst - 2)
    plsc.subcore_barrier()

    pltpu.sync_copy(agg_sh.at[pl.ds(zbase, ROWS_PER_TILE)],
                    agg_out.at[c].at[pl.ds(zbase, ROWS_PER_TILE)])
    if do_cnt:
        pltpu.sync_copy(cnt_sh.at[pl.ds(s * CNT_PER_TILE, CNT_PER_TILE)],
                        cnt_out.at[c].at[pl.ds(s * CNT_PER_TILE, CNT_PER_TILE)])


def _make_sc_agg(do_cnt):
    mesh = plsc.VectorSubcoreMesh(core_axis_name="c", subcore_axis_name="s")
    out_type = [jax.ShapeDtypeStruct((N_SC, N_PAD, D), jnp.float32)]
    if do_cnt:
        out_type.append(jax.ShapeDtypeStruct((N_SC, NC_PAD), jnp.float32))
    scratch = [
        pltpu.VMEM((SS0, CHUNK), jnp.int32),
        pltpu.VMEM((SS0, CHUNK), jnp.int32),
        pltpu.VMEM((CHUNK, D), jnp.float32),
        pltpu.VMEM((CHUNK, D), jnp.float32),
    ]
    if do_cnt:
        scratch.append(pltpu.VMEM((CHUNK,), jnp.float32))
    scratch.append(pltpu.VMEM_SHARED((N_PAD, D), jnp.float32))
    if do_cnt:
        scratch.append(pltpu.VMEM_SHARED((NC_PAD,), jnp.float32))
    scratch.append(pltpu.SemaphoreType.DMA)
    return pl.kernel(
        functools.partial(_sc_agg_body, do_cnt),
        out_type=tuple(out_type) if len(out_type) > 1 else out_type[0],
        mesh=mesh,
        scratch_types=tuple(scratch),
    )


_sc_agg_cnt = _make_sc_agg(True)
_sc_agg = _make_sc_agg(False)

BR = 1000


def _dense_body(agg_ref, cnt_ref, h_ref, wl_ref, wr_ref, b_ref, a_ref, out_ref):
    r = agg_ref[0] + agg_ref[1]
    cnt = cnt_ref[0] + cnt_ref[1]
    r = r / jnp.maximum(cnt, 1.0)
    o = (jnp.dot(r, wl_ref[...], preferred_element_type=jnp.float32)
         + b_ref[...]
         + jnp.dot(h_ref[...], wr_ref[...], preferred_element_type=jnp.float32))
    av = a_ref[0, 0]
    out_ref[...] = jnp.where(o >= 0, o, av * o)


_dense = pl.pallas_call(
    _dense_body,
    grid=(N_NODES // BR,),
    in_specs=[
        pl.BlockSpec((N_SC, BR, D), lambda i: (0, i, 0)),
        pl.BlockSpec((N_SC, BR, 1), lambda i: (0, i, 0)),
        pl.BlockSpec((BR, D), lambda i: (i, 0)),
        pl.BlockSpec((D, D), lambda i: (0, 0)),
        pl.BlockSpec((D, D), lambda i: (0, 0)),
        pl.BlockSpec((1, D), lambda i: (0, 0)),
        pl.BlockSpec((1, 1), lambda i: (0, 0)),
    ],
    out_specs=pl.BlockSpec((BR, D), lambda i: (i, 0)),
    out_shape=jax.ShapeDtypeStruct((N_NODES, D), jnp.float32),
)


_dense_win = pl.pallas_call(
    _dense_body,
    grid=(1,),
    in_specs=[
        pl.BlockSpec((N_SC, 1024, D), lambda i: (0, 0, 0)),
        pl.BlockSpec((N_SC, 1024, 1), lambda i: (0, 0, 0)),
        pl.BlockSpec((1024, D), lambda i: (0, 0)),
        pl.BlockSpec((D, D), lambda i: (0, 0)),
        pl.BlockSpec((D, D), lambda i: (0, 0)),
        pl.BlockSpec((1, D), lambda i: (0, 0)),
        pl.BlockSpec((1, 1), lambda i: (0, 0)),
    ],
    out_specs=pl.BlockSpec((1024, D), lambda i: (0, 0)),
    out_shape=jax.ShapeDtypeStruct((1024, D), jnp.float32),
)


def kernel(x, edge_index, W1_l, b1, W1_r, W2_l, b2, W2_r, W3_l, b3, W3_r, a,
           batch_size, layer):
    src = edge_index[0]
    dst = edge_index[1]
    pad = N_CHUNKS_PAD * CHUNK - E_EDGES
    srcm = jnp.concatenate([src, jnp.zeros((pad,), jnp.int32)]).reshape(N_CHUNKS_PAD, CHUNK)
    dstm = jnp.concatenate([dst, jnp.full((pad,), N_NODES, jnp.int32)]).reshape(N_CHUNKS_PAD, CHUNK)
    a2 = jnp.reshape(a, (1, 1)).astype(jnp.float32)

    agg1, cnt = _sc_agg_cnt(x, srcm, dstm)
    cnt3 = cnt[:, :, None]
    h1 = _dense(agg1, cnt3, x, W1_l, W1_r, b1.reshape(1, D), a2)
    agg2 = _sc_agg(h1, srcm, dstm)
    h2 = _dense(agg2, cnt3, h1, W2_l, W2_r, b2.reshape(1, D), a2)
    agg3 = _sc_agg(h2, srcm, dstm)
    lo = jnp.clip(jnp.asarray(batch_size, jnp.int32) - 1024, 0, N_NODES - 1024)
    agg3w = lax.dynamic_slice(agg3, (0, lo, 0), (N_SC, 1024, D))
    cnt3w = lax.dynamic_slice(cnt3, (0, lo, 0), (N_SC, 1024, 1))
    h2w = lax.dynamic_slice(h2, (lo, 0), (1024, D))
    return _dense_win(agg3w, cnt3w, h2w, W3_l, W3_r, b3.reshape(1, D), a2)
